# Initial kernel scaffold; baseline (speedup 1.0000x reference)
import jax
import jax.numpy as jnp
from jax.experimental import pallas as pl


def kernel(x, W_enc, b_enc, W_e, b_e, inv_temp):
    raise NotImplementedError("write your pallas kernel here")


# speedup vs baseline: 6.6297x; 6.6297x over previous
import jax
import jax.numpy as jnp
from jax.experimental import pallas as pl

NUM_VISIBLE = 1024
NUM_EXPERTS = 16
HIDDEN_DIM = 256
ENERGY_SCALE = 3.0

BT = 512


def _fused_kernel(x_ref, wenc_ref, benc_ref, wh_ref, t_ref, beta_ref,
                  p_ref, e_ref, l_ref):
    enc = jnp.tanh(
        jnp.dot(x_ref[...], wenc_ref[...], preferred_element_type=jnp.float32)
        + benc_ref[...])
    s = jnp.dot(enc, wh_ref[...], preferred_element_type=jnp.float32)
    en = ENERGY_SCALE * jnp.tanh(s + t_ref[...])
    lg = (-beta_ref[0, 0]) * en
    m = jnp.max(lg, axis=-1, keepdims=True)
    ex = jnp.exp(lg - m)
    p_ref[...] = ex / jnp.sum(ex, axis=-1, keepdims=True)
    e_ref[...] = en
    l_ref[...] = lg


def kernel(x, W_enc, b_enc, W_e, b_e, inv_temp):
    B = x.shape[0]
    H = HIDDEN_DIM
    E = NUM_EXPERTS
    w_h = W_e[:H]
    t = (W_e[H:, 0] + b_e).reshape(1, E)
    beta = jax.nn.softplus(inv_temp).reshape(1, 1)
    b_enc2 = b_enc.reshape(1, H)

    grid = (B // BT,)
    out_shape = [jax.ShapeDtypeStruct((B, E), jnp.float32)] * 3
    probs, energies, logits = pl.pallas_call(
        _fused_kernel,
        grid=grid,
        in_specs=[
            pl.BlockSpec((BT, NUM_VISIBLE), lambda i: (i, 0)),
            pl.BlockSpec((NUM_VISIBLE, H), lambda i: (0, 0)),
            pl.BlockSpec((1, H), lambda i: (0, 0)),
            pl.BlockSpec((H, 1), lambda i: (0, 0)),
            pl.BlockSpec((1, E), lambda i: (0, 0)),
            pl.BlockSpec((1, 1), lambda i: (0, 0)),
        ],
        out_specs=[pl.BlockSpec((BT, E), lambda i: (i, 0))] * 3,
        out_shape=out_shape,
    )(x, W_enc, b_enc2, w_h, t, beta)
    return (probs, energies, logits)
